# baseline (speedup 1.0000x reference)
import functools

import jax
import jax.numpy as jnp
from jax import lax
from jax.experimental import pallas as pl
from jax.experimental.pallas import tpu as pltpu
from jax.experimental.pallas import tpu_sc as plsc

N = 10000
D = 128
E = 320000
NC = 2
NS = 16
NW = NC * NS
CHUNK = 128
CHUNKS = 80
E_PAD = NW * CHUNKS * CHUNK
TOT_CHUNKS = NW * CHUNKS
K0 = 104
K1 = 160 - K0
KMAX2 = max(K0, K1)
DUMP = N
N_PAD = 10240
PER_SUB = N_PAD // NS

_MESH = plsc.VectorSubcoreMesh(core_axis_name="c", subcore_axis_name="s")


DEG_ROWS = N_PAD // CHUNK
DEG_PER_SUB = 8


def _deg_body(dst_hbm, iota_hbm, zeros_hbm, deg_hbm,
              dst_v, iota_v, hist_v, deg_sh):
    c = lax.axis_index("c")
    s = lax.axis_index("s")
    wid = c * NS + s
    zero16 = jnp.zeros((16,), jnp.float32)

    def zbody(i, carry):
        for l in range(CHUNK // 16):
            hist_v[i, pl.ds(l * 16, 16)] = zero16
        return carry

    lax.fori_loop(0, DEG_ROWS, zbody, 0)
    @pl.when(s < DEG_ROWS // DEG_PER_SUB)
    def _():
        pltpu.sync_copy(zeros_hbm, deg_sh.at[pl.ds(s * DEG_PER_SUB, DEG_PER_SUB)])
    pltpu.sync_copy(dst_hbm.at[wid], dst_v)
    pltpu.sync_copy(iota_hbm, iota_v)
    ones16 = jnp.full((16,), 1.0, jnp.float32)

    def body(j, carry):
        for l in range(CHUNK // 16):
            idx16 = dst_v[j, pl.ds(l * 16, 16)]
            r = lax.shift_right_logical(idx16, 7)
            col = lax.bitwise_and(idx16, 127)
            plsc.addupdate_scatter(hist_v, [r, col], ones16)
        return carry

    lax.fori_loop(0, CHUNKS, body, 0)
    plsc.subcore_barrier()
    pltpu.sync_copy(hist_v, deg_sh.at[iota_v.at[0]], add=True)
    plsc.subcore_barrier()

    @pl.when(s < DEG_ROWS // DEG_PER_SUB)
    def _():
        sl = pl.ds(s * DEG_PER_SUB, DEG_PER_SUB)
        pltpu.sync_copy(deg_sh.at[sl],
                        deg_hbm.at[pl.ds(c * DEG_ROWS + s * DEG_PER_SUB, DEG_PER_SUB)])


_deg_call = functools.partial(
    pl.kernel,
    out_type=jax.ShapeDtypeStruct((NC * DEG_ROWS, CHUNK), jnp.float32),
    mesh=_MESH,
    scratch_types=[
        pltpu.VMEM((CHUNKS, CHUNK), jnp.int32),
        pltpu.VMEM((1, DEG_ROWS), jnp.int32),
        pltpu.VMEM((DEG_ROWS, CHUNK), jnp.float32),
        pltpu.VMEM_SHARED((DEG_ROWS, CHUNK), jnp.float32),
    ],
    compiler_params=pltpu.CompilerParams(needs_layout_passes=False),
)(_deg_body)


def _edge_body(src_hbm, dst_hbm, h2_hbm, zeros_hbm, acc_hbm,
               src_v, dst_v, rows_a, acc_sh, sem_a):
    c = lax.axis_index("c")
    s = lax.axis_index("s")
    for k in range(PER_SUB // CHUNK):
        pltpu.sync_copy(zeros_hbm, acc_sh.at[pl.ds(s * PER_SUB + k * CHUNK, CHUNK)])
    plsc.subcore_barrier()

    base = lax.select(c == 0, s * K0, NS * K0 + s * K1)
    k = lax.select(c == 0, K0, K1)
    pltpu.sync_copy(src_hbm.at[pl.ds(base, KMAX2)], src_v)
    pltpu.sync_copy(dst_hbm.at[pl.ds(base, KMAX2)], dst_v)

    def body(j, carry):
        pltpu.async_copy(h2_hbm.at[src_v.at[j]], rows_a, sem_a).wait()
        pltpu.sync_copy(rows_a, acc_sh.at[dst_v.at[j]], add=True)
        return carry

    lax.fori_loop(0, k, body, 0)
    plsc.subcore_barrier()
    sl = pl.ds(s * PER_SUB, PER_SUB)
    pltpu.sync_copy(acc_sh.at[sl], acc_hbm.at[pl.ds(c * N_PAD + s * PER_SUB, PER_SUB)])


_edge_call = functools.partial(
    pl.kernel,
    out_type=jax.ShapeDtypeStruct((NC * N_PAD, D), jnp.float32),
    mesh=_MESH,
    scratch_types=[
        pltpu.VMEM((KMAX2, CHUNK), jnp.int32),
        pltpu.VMEM((KMAX2, CHUNK), jnp.int32),
        pltpu.VMEM((CHUNK, D), jnp.float32),
        pltpu.VMEM_SHARED((N_PAD, D), jnp.float32),
        pltpu.SemaphoreType.DMA,
    ],
)(_edge_body)


def _h2_body(x_ref, w_ref, d0_ref, d1_ref, o_ref):
    deg = d0_ref[...] + d1_ref[...] + 1.0
    dinv = lax.rsqrt(deg)
    h = jnp.dot(x_ref[...], w_ref[...], preferred_element_type=jnp.float32)
    o_ref[...] = h * dinv


_h2_call = pl.pallas_call(
    _h2_body,
    out_shape=jax.ShapeDtypeStruct((N, D), jnp.float32),
)


def _bn_body(h2_ref, a0_ref, a1_ref, d0_ref, d1_ref, g_ref, bt_ref, o_ref):
    deg = d0_ref[...] + d1_ref[...] + 1.0
    dinv = lax.rsqrt(deg)
    pre = (h2_ref[...] + a0_ref[...] + a1_ref[...]) * dinv
    mean = jnp.mean(pre, axis=0, keepdims=True)
    cent = pre - mean
    var = jnp.mean(cent * cent, axis=0, keepdims=True)
    y = cent * lax.rsqrt(var + 1e-5) * g_ref[...] + bt_ref[...]
    o_ref[...] = jnp.maximum(y, 0.0)


_bn_call = pl.pallas_call(
    _bn_body,
    out_shape=jax.ShapeDtypeStruct((N, D), jnp.float32),
)


def kernel(x, W, b, gamma, beta, edge_index_t):
    del b
    src = edge_index_t[0].astype(jnp.int32)
    dst = edge_index_t[1].astype(jnp.int32)
    pad = E_PAD - E
    src_p = jnp.concatenate([src, jnp.zeros((pad,), jnp.int32)])
    dst_p = jnp.concatenate([dst, jnp.full((pad,), DUMP, jnp.int32)])
    src_g = src_p.reshape(NW, CHUNKS, CHUNK)
    dst_g = dst_p.reshape(NW, CHUNKS, CHUNK)
    zpad = jnp.zeros((KMAX2 * CHUNK,), jnp.int32)
    src_e = jnp.concatenate([src_p, zpad]).reshape(TOT_CHUNKS + KMAX2, CHUNK)
    dst_e = jnp.concatenate([dst_p, jnp.full((KMAX2 * CHUNK,), DUMP, jnp.int32)]
                            ).reshape(TOT_CHUNKS + KMAX2, CHUNK)
    iota80 = jnp.arange(DEG_ROWS, dtype=jnp.int32).reshape(1, DEG_ROWS)
    zeros5 = jnp.zeros((DEG_PER_SUB, CHUNK), jnp.float32)
    zerosD = jnp.zeros((CHUNK, D), jnp.float32)

    degp = _deg_call(dst_g, iota80, zeros5)
    d0 = degp[:DEG_ROWS].reshape(N_PAD, 1)[:N]
    d1 = degp[DEG_ROWS:].reshape(N_PAD, 1)[:N]
    h2 = _h2_call(x, W, d0, d1)
    accp = _edge_call(src_e, dst_e, h2, zerosD)
    out = _bn_call(h2, accp[:N], accp[N_PAD:N_PAD + N], d0, d1,
                   gamma.reshape(1, D), beta.reshape(1, D))
    return out


# speedup vs baseline: 1.0373x; 1.0373x over previous
import functools

import jax
import jax.numpy as jnp
from jax import lax
from jax.experimental import pallas as pl
from jax.experimental.pallas import tpu as pltpu
from jax.experimental.pallas import tpu_sc as plsc

N = 10000
D = 128
E = 320000
NC = 2
NS = 16
NW = NC * NS
CHUNK = 128
CHUNKS = 80
E_PAD = NW * CHUNKS * CHUNK
TOT_CHUNKS = NW * CHUNKS
K0 = 80
K1 = 160 - K0
KMAX2 = max(K0, K1)
DUMP = N
N_PAD = 10240
PER_SUB = N_PAD // NS

_MESH = plsc.VectorSubcoreMesh(core_axis_name="c", subcore_axis_name="s")


DEG_ROWS = N_PAD // CHUNK
DEG_PER_SUB = 8


def _deg_body(dst_hbm, iota_hbm, zeros_hbm, deg_hbm,
              dst_v, iota_v, hist_v, deg_sh):
    c = lax.axis_index("c")
    s = lax.axis_index("s")
    wid = c * NS + s
    zero16 = jnp.zeros((16,), jnp.float32)

    def zbody(i, carry):
        for l in range(CHUNK // 16):
            hist_v[i, pl.ds(l * 16, 16)] = zero16
        return carry

    lax.fori_loop(0, DEG_ROWS, zbody, 0)
    @pl.when(s < DEG_ROWS // DEG_PER_SUB)
    def _():
        pltpu.sync_copy(zeros_hbm, deg_sh.at[pl.ds(s * DEG_PER_SUB, DEG_PER_SUB)])
    pltpu.sync_copy(dst_hbm.at[wid], dst_v)
    pltpu.sync_copy(iota_hbm, iota_v)
    ones16 = jnp.full((16,), 1.0, jnp.float32)

    def body(j, carry):
        for l in range(CHUNK // 16):
            idx16 = dst_v[j, pl.ds(l * 16, 16)]
            r = lax.shift_right_logical(idx16, 7)
            col = lax.bitwise_and(idx16, 127)
            plsc.addupdate_scatter(hist_v, [r, col], ones16)
        return carry

    lax.fori_loop(0, CHUNKS, body, 0)
    plsc.subcore_barrier()
    pltpu.sync_copy(hist_v, deg_sh.at[iota_v.at[0]], add=True)
    plsc.subcore_barrier()

    @pl.when(s < DEG_ROWS // DEG_PER_SUB)
    def _():
        sl = pl.ds(s * DEG_PER_SUB, DEG_PER_SUB)
        pltpu.sync_copy(deg_sh.at[sl],
                        deg_hbm.at[pl.ds(c * DEG_ROWS + s * DEG_PER_SUB, DEG_PER_SUB)])


_deg_call = functools.partial(
    pl.kernel,
    out_type=jax.ShapeDtypeStruct((NC * DEG_ROWS, CHUNK), jnp.float32),
    mesh=_MESH,
    scratch_types=[
        pltpu.VMEM((CHUNKS, CHUNK), jnp.int32),
        pltpu.VMEM((1, DEG_ROWS), jnp.int32),
        pltpu.VMEM((DEG_ROWS, CHUNK), jnp.float32),
        pltpu.VMEM_SHARED((DEG_ROWS, CHUNK), jnp.float32),
    ],
    compiler_params=pltpu.CompilerParams(needs_layout_passes=False),
)(_deg_body)


def _edge_body(src_hbm, dst_hbm, h2_hbm, zeros_hbm, acc_hbm,
               src_v, dst_v, rows_a, acc_sh, sem_a):
    c = lax.axis_index("c")
    s = lax.axis_index("s")
    for k in range(PER_SUB // CHUNK):
        pltpu.sync_copy(zeros_hbm, acc_sh.at[pl.ds(s * PER_SUB + k * CHUNK, CHUNK)])
    plsc.subcore_barrier()

    base = lax.select(c == 0, s * K0, NS * K0 + s * K1)
    k = lax.select(c == 0, K0, K1)
    pltpu.sync_copy(src_hbm.at[pl.ds(base, KMAX2)], src_v)
    pltpu.sync_copy(dst_hbm.at[pl.ds(base, KMAX2)], dst_v)

    def body(j, carry):
        pltpu.async_copy(h2_hbm.at[src_v.at[j]], rows_a, sem_a).wait()
        pltpu.sync_copy(rows_a, acc_sh.at[dst_v.at[j]], add=True)
        return carry

    lax.fori_loop(0, k, body, 0)
    plsc.subcore_barrier()
    sl = pl.ds(s * PER_SUB, PER_SUB)
    pltpu.sync_copy(acc_sh.at[sl], acc_hbm.at[pl.ds(c * N_PAD + s * PER_SUB, PER_SUB)])


_edge_call = functools.partial(
    pl.kernel,
    out_type=jax.ShapeDtypeStruct((NC * N_PAD, D), jnp.float32),
    mesh=_MESH,
    scratch_types=[
        pltpu.VMEM((KMAX2, CHUNK), jnp.int32),
        pltpu.VMEM((KMAX2, CHUNK), jnp.int32),
        pltpu.VMEM((CHUNK, D), jnp.float32),
        pltpu.VMEM_SHARED((N_PAD, D), jnp.float32),
        pltpu.SemaphoreType.DMA,
    ],
)(_edge_body)


def _h2_body(x_ref, w_ref, d0_ref, d1_ref, o_ref):
    deg = d0_ref[...] + d1_ref[...] + 1.0
    dinv = lax.rsqrt(deg)
    h = jnp.dot(x_ref[...], w_ref[...], preferred_element_type=jnp.float32)
    o_ref[...] = h * dinv


_h2_call = pl.pallas_call(
    _h2_body,
    out_shape=jax.ShapeDtypeStruct((N, D), jnp.float32),
)


def _bn_body(h2_ref, a0_ref, a1_ref, d0_ref, d1_ref, g_ref, bt_ref, o_ref):
    deg = d0_ref[...] + d1_ref[...] + 1.0
    dinv = lax.rsqrt(deg)
    pre = (h2_ref[...] + a0_ref[...] + a1_ref[...]) * dinv
    mean = jnp.mean(pre, axis=0, keepdims=True)
    cent = pre - mean
    var = jnp.mean(cent * cent, axis=0, keepdims=True)
    y = cent * lax.rsqrt(var + 1e-5) * g_ref[...] + bt_ref[...]
    o_ref[...] = jnp.maximum(y, 0.0)


_bn_call = pl.pallas_call(
    _bn_body,
    out_shape=jax.ShapeDtypeStruct((N, D), jnp.float32),
)


def kernel(x, W, b, gamma, beta, edge_index_t):
    del b
    src = edge_index_t[0].astype(jnp.int32)
    dst = edge_index_t[1].astype(jnp.int32)
    pad = E_PAD - E
    src_p = jnp.concatenate([src, jnp.zeros((pad,), jnp.int32)])
    dst_p = jnp.concatenate([dst, jnp.full((pad,), DUMP, jnp.int32)])
    src_g = src_p.reshape(NW, CHUNKS, CHUNK)
    dst_g = dst_p.reshape(NW, CHUNKS, CHUNK)
    zpad = jnp.zeros((KMAX2 * CHUNK,), jnp.int32)
    src_e = jnp.concatenate([src_p, zpad]).reshape(TOT_CHUNKS + KMAX2, CHUNK)
    dst_e = jnp.concatenate([dst_p, jnp.full((KMAX2 * CHUNK,), DUMP, jnp.int32)]
                            ).reshape(TOT_CHUNKS + KMAX2, CHUNK)
    iota80 = jnp.arange(DEG_ROWS, dtype=jnp.int32).reshape(1, DEG_ROWS)
    zeros5 = jnp.zeros((DEG_PER_SUB, CHUNK), jnp.float32)
    zerosD = jnp.zeros((CHUNK, D), jnp.float32)

    degp = _deg_call(dst_g, iota80, zeros5)
    d0 = degp[:DEG_ROWS].reshape(N_PAD, 1)[:N]
    d1 = degp[DEG_ROWS:].reshape(N_PAD, 1)[:N]
    h2 = _h2_call(x, W, d0, d1)
    accp = _edge_call(src_e, dst_e, h2, zerosD)
    out = _bn_call(h2, accp[:N], accp[N_PAD:N_PAD + N], d0, d1,
                   gamma.reshape(1, D), beta.reshape(1, D))
    return out


# speedup vs baseline: 1.5173x; 1.4628x over previous
import functools

import jax
import jax.numpy as jnp
from jax import lax
from jax.experimental import pallas as pl
from jax.experimental.pallas import tpu as pltpu
from jax.experimental.pallas import tpu_sc as plsc

N = 10000
D = 128
E = 320000
NC = 2
NS = 16
NW = NC * NS
CHUNK = 128
CHUNKS = -(-E // (NW * CHUNK))
E_PAD = NW * CHUNKS * CHUNK
DUMP = N
N_PAD = 10240
PER_SUB = N_PAD // NS

_MESH = plsc.VectorSubcoreMesh(core_axis_name="c", subcore_axis_name="s")


DEG_ROWS = N_PAD // CHUNK
DEG_PER_SUB = 8


def _deg_body(dst_hbm, iota_hbm, zeros_hbm, deg_hbm,
              dst_v, iota_v, hist_v, deg_sh):
    c = lax.axis_index("c")
    s = lax.axis_index("s")
    wid = c * NS + s
    zero16 = jnp.zeros((16,), jnp.float32)

    def zbody(i, carry):
        for l in range(CHUNK // 16):
            hist_v[i, pl.ds(l * 16, 16)] = zero16
        return carry

    lax.fori_loop(0, DEG_ROWS, zbody, 0)
    @pl.when(s < DEG_ROWS // DEG_PER_SUB)
    def _():
        pltpu.sync_copy(zeros_hbm, deg_sh.at[pl.ds(s * DEG_PER_SUB, DEG_PER_SUB)])
    pltpu.sync_copy(dst_hbm.at[wid], dst_v)
    pltpu.sync_copy(iota_hbm, iota_v)
    ones16 = jnp.full((16,), 1.0, jnp.float32)

    def body(j, carry):
        for l in range(CHUNK // 16):
            idx16 = dst_v[j, pl.ds(l * 16, 16)]
            r = lax.shift_right_logical(idx16, 7)
            col = lax.bitwise_and(idx16, 127)
            plsc.addupdate_scatter(hist_v, [r, col], ones16)
        return carry

    lax.fori_loop(0, CHUNKS, body, 0)
    plsc.subcore_barrier()
    pltpu.sync_copy(hist_v, deg_sh.at[iota_v.at[0]], add=True)
    plsc.subcore_barrier()

    @pl.when(s < DEG_ROWS // DEG_PER_SUB)
    def _():
        sl = pl.ds(s * DEG_PER_SUB, DEG_PER_SUB)
        pltpu.sync_copy(deg_sh.at[sl],
                        deg_hbm.at[pl.ds(c * DEG_ROWS + s * DEG_PER_SUB, DEG_PER_SUB)])


_deg_call = functools.partial(
    pl.kernel,
    out_type=jax.ShapeDtypeStruct((NC * DEG_ROWS, CHUNK), jnp.float32),
    mesh=_MESH,
    scratch_types=[
        pltpu.VMEM((CHUNKS, CHUNK), jnp.int32),
        pltpu.VMEM((1, DEG_ROWS), jnp.int32),
        pltpu.VMEM((DEG_ROWS, CHUNK), jnp.float32),
        pltpu.VMEM_SHARED((DEG_ROWS, CHUNK), jnp.float32),
    ],
    compiler_params=pltpu.CompilerParams(needs_layout_passes=False),
)(_deg_body)


def _edge_body(src_hbm, dst_hbm, h2_hbm, zeros_hbm, acc_hbm,
               src_v, dst_v, rows_a, acc_sh, sem_a):
    c = lax.axis_index("c")
    s = lax.axis_index("s")
    wid = c * NS + s
    for k in range(PER_SUB // CHUNK):
        pltpu.sync_copy(zeros_hbm, acc_sh.at[pl.ds(s * PER_SUB + k * CHUNK, CHUNK)])
    pltpu.sync_copy(src_hbm.at[wid], src_v)
    pltpu.sync_copy(dst_hbm.at[wid], dst_v)
    plsc.subcore_barrier()

    def body(j, carry):
        pltpu.async_copy(h2_hbm.at[src_v.at[j]], rows_a, sem_a).wait()
        pltpu.sync_copy(rows_a, acc_sh.at[dst_v.at[j]], add=True)
        return carry

    lax.fori_loop(0, CHUNKS, body, 0)
    plsc.subcore_barrier()
    sl = pl.ds(s * PER_SUB, PER_SUB)
    pltpu.sync_copy(acc_sh.at[sl], acc_hbm.at[pl.ds(c * N_PAD + s * PER_SUB, PER_SUB)])


_edge_call = functools.partial(
    pl.kernel,
    out_type=jax.ShapeDtypeStruct((NC * N_PAD, D), jnp.float32),
    mesh=_MESH,
    scratch_types=[
        pltpu.VMEM((CHUNKS, CHUNK), jnp.int32),
        pltpu.VMEM((CHUNKS, CHUNK), jnp.int32),
        pltpu.VMEM((CHUNK, D), jnp.float32),
        pltpu.VMEM_SHARED((N_PAD, D), jnp.float32),
        pltpu.SemaphoreType.DMA,
    ],
)(_edge_body)


def _h2_body(x_ref, w_ref, d0_ref, d1_ref, o_ref):
    deg = d0_ref[...] + d1_ref[...] + 1.0
    dinv = lax.rsqrt(deg)
    h = jnp.dot(x_ref[...], w_ref[...], preferred_element_type=jnp.float32)
    o_ref[...] = h * dinv


_h2_call = pl.pallas_call(
    _h2_body,
    out_shape=jax.ShapeDtypeStruct((N, D), jnp.float32),
)


def _bn_body(h2_ref, a0_ref, a1_ref, d0_ref, d1_ref, g_ref, bt_ref, o_ref):
    deg = d0_ref[...] + d1_ref[...] + 1.0
    dinv = lax.rsqrt(deg)
    pre = (h2_ref[...] + a0_ref[...] + a1_ref[...]) * dinv
    mean = jnp.mean(pre, axis=0, keepdims=True)
    cent = pre - mean
    var = jnp.mean(cent * cent, axis=0, keepdims=True)
    y = cent * lax.rsqrt(var + 1e-5) * g_ref[...] + bt_ref[...]
    o_ref[...] = jnp.maximum(y, 0.0)


_bn_call = pl.pallas_call(
    _bn_body,
    out_shape=jax.ShapeDtypeStruct((N, D), jnp.float32),
)


def kernel(x, W, b, gamma, beta, edge_index_t):
    del b
    src = edge_index_t[0].astype(jnp.int32)
    dst = edge_index_t[1].astype(jnp.int32)
    pad = E_PAD - E
    src_g = jnp.concatenate([src, jnp.zeros((pad,), jnp.int32)]).reshape(NW, CHUNKS, CHUNK)
    dst_g = jnp.concatenate([dst, jnp.full((pad,), DUMP, jnp.int32)]).reshape(NW, CHUNKS, CHUNK)
    iota80 = jnp.arange(DEG_ROWS, dtype=jnp.int32).reshape(1, DEG_ROWS)
    zeros5 = jnp.zeros((DEG_PER_SUB, CHUNK), jnp.float32)
    zerosD = jnp.zeros((CHUNK, D), jnp.float32)

    degp = _deg_call(dst_g, iota80, zeros5)
    d0 = degp[:DEG_ROWS].reshape(N_PAD, 1)[:N]
    d1 = degp[DEG_ROWS:].reshape(N_PAD, 1)[:N]
    h2 = _h2_call(x, W, d0, d1)
    accp = _edge_call(src_g, dst_g, h2, zerosD)
    out = _bn_call(h2, accp[:N], accp[N_PAD:N_PAD + N], d0, d1,
                   gamma.reshape(1, D), beta.reshape(1, D))
    return out
